# baseline (speedup 1.0000x reference)
import jax
import jax.numpy as jnp
from jax import lax
from jax.experimental import pallas as pl
from jax.experimental.pallas import tpu as pltpu
from jax.experimental.pallas import tpu_sc as plsc

N_NODES = 32768
N_EDGES = 1048576
N_GRAPHS = 128
IN_DIM = 4
EMB = 32
EPS = 1e-5

NC, NS, L = 2, 16, 16
NW = NC * NS
EPW = N_EDGES // NW
EB = 2048
NBLK = EPW // EB
NSUB = EB // 128
NPT = N_NODES // NS

HIGHEST = lax.Precision.HIGHEST


XD = 8


def _sc_agg_body(x_hbm, srch_hbm, parc_hbm, dst_hbm, attr_hbm, zeros_hbm,
                 part_hbm,
                 acc_s,
                 src2_a, attr_a, rows_a, msg_a, parc_a,
                 src2_b, attr_b, rows_b, msg_b, parc_b,
                 dst2_a, dst2_b, dst2_c, trans,
                 isem_a, gsem_a, ssem_a, isem_b, gsem_b, ssem_b):
    c = lax.axis_index("c")
    s = lax.axis_index("s")
    wid = c * NS + s

    bufs = [(src2_a, attr_a, rows_a, msg_a, parc_a, isem_a, gsem_a, ssem_a),
            (src2_b, attr_b, rows_b, msg_b, parc_b, isem_b, gsem_b, ssem_b)]
    dsts = [dst2_a, dst2_b, dst2_c]

    pltpu.sync_copy(zeros_hbm, rows_a)
    pltpu.sync_copy(rows_a, acc_s.at[pl.ds(s * NPT, NPT)])
    pltpu.sync_copy(zeros_hbm, msg_a)
    pltpu.sync_copy(zeros_hbm, msg_b)
    plsc.subcore_barrier()

    iota = lax.iota(jnp.int32, L)
    div4 = lax.shift_right_logical(iota, 2)
    mod4 = lax.bitwise_and(iota, 3)

    base = wid * EPW

    def stage_idx(bi, buf):
        src2, attr_v, _, _, parc, isem, _, _ = buf
        dst2 = dsts[bi % 3]
        off = pl.multiple_of(base + bi * EB, EB)
        return [pltpu.async_copy(srch_hbm.at[pl.ds(off, EB)], src2, isem),
                pltpu.async_copy(parc_hbm.at[pl.ds(off, EB)], parc, isem),
                pltpu.async_copy(dst_hbm.at[pl.ds(off, EB)], dst2, isem),
                pltpu.async_copy(attr_hbm.at[pl.ds(off, EB)], attr_v, isem)]

    def mul_loop(buf):
        _, attr_v, rows_v, msg_v, parc, _, _, _ = buf

        def mgroup(g, _):
            eidx = g * 4 + div4
            p = plsc.load_gather(parc, [eidx])
            a = plsc.load_gather(attr_v, [eidx])
            v = plsc.load_gather(rows_v, [eidx, mod4 + p])
            plsc.store_scatter(msg_v, [eidx, mod4], v * a)
            return 0

        lax.fori_loop(0, EB // 4, mgroup, 0, unroll=8)

    idx_d = [None, None]
    gat_d = [None, None]
    sca_d = [None, None]
    idx_d[0] = stage_idx(0, bufs[0])
    for bi in range(NBLK):
        cur = bi % 2
        nxt = 1 - cur
        src2, _, rows_v, msg_v, _, _, gsem, ssem = bufs[cur]
        for d in idx_d[cur]:
            d.wait()
        gat_d[cur] = [pltpu.async_copy(x_hbm.at[src2.at[pl.ds(j * 128, 128)]],
                                       rows_v.at[pl.ds(j * 128, 128)], gsem)
                      for j in range(NSUB)]
        if sca_d[cur] is not None:
            for d in sca_d[cur]:
                d.wait()
            sca_d[cur] = None
        if bi + 1 < NBLK:
            idx_d[nxt] = stage_idx(bi + 1, bufs[nxt])
        for d in gat_d[cur]:
            d.wait()
        mul_loop(bufs[cur])
        sca_d[cur] = [pltpu.async_copy(
                          msg_v.at[pl.ds(j * 128, 128)],
                          acc_s.at[dsts[bi % 3].at[pl.ds(j * 128, 128)]],
                          ssem, add=True)
                      for j in range(NSUB)]
    for p in range(2):
        if sca_d[p] is not None:
            for d in sca_d[p]:
                d.wait()
    plsc.subcore_barrier()

    pltpu.sync_copy(acc_s.at[pl.ds(s * NPT, NPT)], rows_a)

    def tpose(i, _):
        f = lax.shift_right_logical(i, 7)
        n = lax.bitwise_and(i, 127) * L
        nidx = n + iota
        fvec = jnp.full((L,), f, jnp.int32)
        v = plsc.load_gather(rows_a, [nidx, fvec])
        plsc.store_scatter(trans, [fvec,
                                   lax.shift_right_logical(nidx, 7),
                                   lax.bitwise_and(nidx, 127)], v)
        return 0

    lax.fori_loop(0, XD * (NPT // L), tpose, 0, unroll=4)
    pltpu.sync_copy(trans,
                    part_hbm.at[pl.ds(c * XD, XD), pl.ds(s * (NPT // 128),
                                                         NPT // 128)])


def _sc_aggregate(x8, srch, parc, dst, attr, zeros):
    mesh = plsc.VectorSubcoreMesh(core_axis_name="c", subcore_axis_name="s")
    fn = pl.kernel(
        _sc_agg_body,
        out_type=jax.ShapeDtypeStruct((NC * XD, N_NODES // 128, 128),
                                      jnp.float32),
        mesh=mesh,
        scratch_types=(
            [pltpu.VMEM_SHARED((N_NODES, XD), jnp.float32)]
            + 2 * [pltpu.VMEM((EB,), jnp.int32),
                   pltpu.VMEM((EB,), jnp.float32),
                   pltpu.VMEM((EB, XD), jnp.float32),
                   pltpu.VMEM((EB, XD), jnp.float32),
                   pltpu.VMEM((EB,), jnp.int32)]
            + 3 * [pltpu.VMEM((EB,), jnp.int32)]
            + [pltpu.VMEM((XD, NPT // 128, 128), jnp.float32)]
            + 6 * [pltpu.SemaphoreType.DMA]
        ),
        compiler_params=pltpu.CompilerParams(use_tc_tiling_on_sc=False,
                                             needs_layout_passes=False),
    )
    return fn(x8, srch, parc, dst, attr, zeros)


TN = 2048
NST = N_NODES // TN


def _tc_stats_body(p_ref, w_ref, b_ref, bidx_ref, h_ref, s12_ref, cnt_ref):
    i = pl.program_id(0)
    p = p_ref[...].reshape(NC * XD, TN)
    ps = p[:XD] + p[XD:]
    hT = lax.dot_general(w_ref[...], ps, (((0,), (0,)), ((), ())),
                         preferred_element_type=jnp.float32)
    hT = jnp.maximum(hT + b_ref[...], 0.0)
    h_ref[...] = hT.reshape(EMB, TN // 128, 128)
    bidx = bidx_ref[0, 0]
    oh = jnp.equal(bidx[:, None],
                   lax.broadcasted_iota(jnp.int32, (TN, N_GRAPHS), 1)
                   ).astype(jnp.float32)
    hh = jnp.concatenate([hT, hT * hT], axis=0)
    hi = hh.astype(jnp.bfloat16).astype(jnp.float32)
    lo = hh - hi
    s12 = (lax.dot_general(hi, oh, (((1,), (0,)), ((), ())),
                           preferred_element_type=jnp.float32)
           + lax.dot_general(lo, oh, (((1,), (0,)), ((), ())),
                             preferred_element_type=jnp.float32))
    cnt = jnp.sum(oh, axis=0)[None, :]

    @pl.when(i == 0)
    def _init():
        s12_ref[...] = s12
        cnt_ref[...] = cnt

    @pl.when(i != 0)
    def _acc():
        s12_ref[...] += s12
        cnt_ref[...] += cnt


def _tc_stats(part, w, bcol, bidx3):
    return pl.pallas_call(
        _tc_stats_body,
        grid=(NST,),
        in_specs=[
            pl.BlockSpec((NC * XD, TN // 128, 128), lambda i: (0, i, 0)),
            pl.BlockSpec((XD, EMB), lambda i: (0, 0)),
            pl.BlockSpec((EMB, 1), lambda i: (0, 0)),
            pl.BlockSpec((1, 1, TN), lambda i: (i, 0, 0)),
        ],
        out_specs=[
            pl.BlockSpec((EMB, TN // 128, 128), lambda i: (0, i, 0)),
            pl.BlockSpec((2 * EMB, N_GRAPHS), lambda i: (0, 0)),
            pl.BlockSpec((1, N_GRAPHS), lambda i: (0, 0)),
        ],
        out_shape=[
            jax.ShapeDtypeStruct((EMB, N_NODES // 128, 128), jnp.float32),
            jax.ShapeDtypeStruct((2 * EMB, N_GRAPHS), jnp.float32),
            jax.ShapeDtypeStruct((1, N_GRAPHS), jnp.float32),
        ],
    )(part, w, bcol, bidx3)


GN = N_NODES // N_GRAPHS
DB = 8
DBN = DB * GN


def _tc_dec_body(h_ref, bidx_ref, s12_ref, cnt_ref, gw_ref, gb_ref,
                 gs_ref, out_ref):
    rcnt = 1.0 / jnp.maximum(cnt_ref[...], 1.0)
    meanT = s12_ref[:EMB] * rcnt
    meansqT = s12_ref[EMB:] * rcnt
    sc = gs_ref[...]
    m2 = meanT * meanT
    varT = meansqT - (2.0 * sc) * m2 + (sc * sc) * m2
    stdT = jnp.sqrt(varT + EPS)
    scmeanT = meanT * sc
    hT = h_ref[...].reshape(EMB, DBN)
    gw = gw_ref[...]
    gb = gb_ref[...]
    for g in range(DB):
        bidx_g = bidx_ref[g, 0]
        ohT = jnp.equal(bidx_g[None, :],
                        lax.broadcasted_iota(jnp.int32, (N_GRAPHS, GN), 0)
                        ).astype(jnp.float32)
        meanT_r = lax.dot_general(scmeanT, ohT, (((1,), (0,)), ((), ())),
                                  preferred_element_type=jnp.float32)
        stdT_r = lax.dot_general(stdT, ohT, (((1,), (0,)), ((), ())),
                                 preferred_element_type=jnp.float32)
        zg = gw * (hT[:, g * GN:(g + 1) * GN] - meanT_r) / stdT_r + gb
        a = lax.dot_general(zg, zg, (((0,), (0,)), ((), ())),
                            preferred_element_type=jnp.float32)
        out_ref[g] = jax.nn.sigmoid(a)


def _tc_decode(h, bidx3, s12, cnt, gwc, gbc, gsc):
    return pl.pallas_call(
        _tc_dec_body,
        grid=(N_GRAPHS // DB,),
        in_specs=[
            pl.BlockSpec((EMB, DBN // 128, 128), lambda i: (0, i, 0)),
            pl.BlockSpec((DB, 1, GN), lambda i: (i, 0, 0)),
            pl.BlockSpec((2 * EMB, N_GRAPHS), lambda i: (0, 0)),
            pl.BlockSpec((1, N_GRAPHS), lambda i: (0, 0)),
            pl.BlockSpec((EMB, 1), lambda i: (0, 0)),
            pl.BlockSpec((EMB, 1), lambda i: (0, 0)),
            pl.BlockSpec((EMB, 1), lambda i: (0, 0)),
        ],
        out_specs=pl.BlockSpec((DB, GN, GN), lambda i: (i, 0, 0)),
        out_shape=jax.ShapeDtypeStruct((N_GRAPHS, GN, GN), jnp.float32),
    )(h, bidx3, s12, cnt, gwc, gbc, gsc)


def kernel(x, edge_index, edge_attr, batch_index, W, b, gn_weight, gn_bias,
           gn_mean_scale):
    xr = x.reshape(N_NODES // 2, XD)
    src = edge_index[0]
    srch = lax.shift_right_logical(src, 1)
    parc = lax.bitwise_and(src, 1) * 4
    zeros = jnp.zeros((NPT, XD), jnp.float32)
    part = _sc_aggregate(xr, srch, parc, edge_index[1], edge_attr, zeros)

    bidx_a = batch_index.reshape(NST, 1, TN)
    w8 = jnp.pad(W, ((0, XD - IN_DIM), (0, 0)))
    h, s12, cnt = _tc_stats(part, w8, b.reshape(EMB, 1), bidx_a)

    bidx_b = batch_index.reshape(N_GRAPHS, 1, GN)
    adj = _tc_decode(h, bidx_b, s12, cnt,
                     gn_weight.reshape(EMB, 1), gn_bias.reshape(EMB, 1),
                     gn_mean_scale.reshape(EMB, 1))
    return adj.reshape(N_GRAPHS, 1, GN, GN)


# speedup vs baseline: 1.0723x; 1.0723x over previous
import jax
import jax.numpy as jnp
from jax import lax
from jax.experimental import pallas as pl
from jax.experimental.pallas import tpu as pltpu
from jax.experimental.pallas import tpu_sc as plsc

N_NODES = 32768
N_EDGES = 1048576
N_GRAPHS = 128
IN_DIM = 4
EMB = 32
EPS = 1e-5

NC, NS, L = 2, 16, 16
NW = NC * NS
EPW = N_EDGES // NW
EB = 2048
NBLK = EPW // EB
NSUB = EB // 128
NPT = N_NODES // NS

HIGHEST = lax.Precision.HIGHEST


XD = 8


def _sc_agg_body(x_hbm, ei_hbm, attr_hbm, zeros_hbm, part_hbm,
                 acc_s,
                 src2_a, attr_a, rows_a, msg_a,
                 src2_b, attr_b, rows_b, msg_b,
                 dst2_a, dst2_b, dst2_c, trans,
                 isem_a, gsem_a, ssem_a, isem_b, gsem_b, ssem_b):
    c = lax.axis_index("c")
    s = lax.axis_index("s")
    wid = c * NS + s

    bufs = [(src2_a, attr_a, rows_a, msg_a, isem_a, gsem_a, ssem_a),
            (src2_b, attr_b, rows_b, msg_b, isem_b, gsem_b, ssem_b)]
    dsts = [dst2_a, dst2_b, dst2_c]

    pltpu.sync_copy(zeros_hbm, rows_a)
    pltpu.sync_copy(rows_a, acc_s.at[pl.ds(s * NPT, NPT)])
    plsc.subcore_barrier()

    iota = lax.iota(jnp.int32, L)
    div8 = lax.shift_right_logical(iota, 3)
    mod8 = lax.bitwise_and(iota, 7)

    base = wid * EPW

    def stage_idx(bi, buf):
        src2, attr_v, _, _, isem, _, _ = buf
        dst2 = dsts[bi % 3]
        off = pl.multiple_of(base + bi * EB, EB)
        return [pltpu.async_copy(ei_hbm.at[0, pl.ds(off, EB)], src2, isem),
                pltpu.async_copy(ei_hbm.at[1, pl.ds(off, EB)], dst2, isem),
                pltpu.async_copy(attr_hbm.at[pl.ds(off, EB)], attr_v, isem)]

    def mul_loop(buf):
        _, attr_v, rows_v, msg_v, _, _, _ = buf

        def mgroup(g, _):
            ridx = g * 2 + div8
            a = plsc.load_gather(attr_v, [ridx])
            v = plsc.load_gather(rows_v, [ridx, mod8])
            plsc.store_scatter(msg_v, [ridx, mod8], v * a)
            return 0

        lax.fori_loop(0, EB // 2, mgroup, 0, unroll=8)

    idx_d = [None, None]
    gat_d = [None, None]
    sca_d = [None, None]
    idx_d[0] = stage_idx(0, bufs[0])
    for bi in range(NBLK):
        cur = bi % 2
        nxt = 1 - cur
        src2, _, rows_v, msg_v, _, gsem, ssem = bufs[cur]
        for d in idx_d[cur]:
            d.wait()
        gat_d[cur] = [pltpu.async_copy(x_hbm.at[src2.at[pl.ds(j * 128, 128)]],
                                       rows_v.at[pl.ds(j * 128, 128)], gsem)
                      for j in range(NSUB)]
        if sca_d[cur] is not None:
            for d in sca_d[cur]:
                d.wait()
            sca_d[cur] = None
        if bi + 1 < NBLK:
            idx_d[nxt] = stage_idx(bi + 1, bufs[nxt])
        for d in gat_d[cur]:
            d.wait()
        mul_loop(bufs[cur])
        sca_d[cur] = [pltpu.async_copy(
                          msg_v.at[pl.ds(j * 128, 128)],
                          acc_s.at[dsts[bi % 3].at[pl.ds(j * 128, 128)]],
                          ssem, add=True)
                      for j in range(NSUB)]
    for p in range(2):
        if sca_d[p] is not None:
            for d in sca_d[p]:
                d.wait()
    plsc.subcore_barrier()

    pltpu.sync_copy(acc_s.at[pl.ds(s * NPT, NPT)], rows_a)

    def tpose(i, _):
        f = lax.shift_right_logical(i, 7)
        n = lax.bitwise_and(i, 127) * L
        nidx = n + iota
        fvec = jnp.full((L,), f, jnp.int32)
        v = plsc.load_gather(rows_a, [nidx, fvec])
        plsc.store_scatter(trans, [fvec,
                                   lax.shift_right_logical(nidx, 7),
                                   lax.bitwise_and(nidx, 127)], v)
        return 0

    lax.fori_loop(0, XD * (NPT // L), tpose, 0, unroll=4)
    pltpu.sync_copy(trans,
                    part_hbm.at[pl.ds(c * XD, XD), pl.ds(s * (NPT // 128),
                                                         NPT // 128)])


def _sc_aggregate(x8, ei, attr, zeros):
    mesh = plsc.VectorSubcoreMesh(core_axis_name="c", subcore_axis_name="s")
    fn = pl.kernel(
        _sc_agg_body,
        out_type=jax.ShapeDtypeStruct((NC * XD, N_NODES // 128, 128),
                                      jnp.float32),
        mesh=mesh,
        scratch_types=(
            [pltpu.VMEM_SHARED((N_NODES, XD), jnp.float32)]
            + 2 * [pltpu.VMEM((EB,), jnp.int32),
                   pltpu.VMEM((EB,), jnp.float32),
                   pltpu.VMEM((EB, XD), jnp.float32),
                   pltpu.VMEM((EB, XD), jnp.float32)]
            + 3 * [pltpu.VMEM((EB,), jnp.int32)]
            + [pltpu.VMEM((XD, NPT // 128, 128), jnp.float32)]
            + 6 * [pltpu.SemaphoreType.DMA]
        ),
        compiler_params=pltpu.CompilerParams(use_tc_tiling_on_sc=False,
                                             needs_layout_passes=False),
    )
    return fn(x8, ei, attr, zeros)


TN = 2048
NST = N_NODES // TN


def _tc_stats_body(p_ref, w_ref, b_ref, bidx_ref, h_ref, s12_ref, cnt_ref):
    i = pl.program_id(0)
    p = p_ref[...].reshape(NC * XD, TN)
    ps = p[:XD] + p[XD:]
    hT = lax.dot_general(w_ref[...], ps, (((0,), (0,)), ((), ())),
                         preferred_element_type=jnp.float32)
    hT = jnp.maximum(hT + b_ref[...], 0.0)
    h_ref[...] = hT.reshape(EMB, TN // 128, 128)
    bidx = bidx_ref[0, 0]
    oh = jnp.equal(bidx[:, None],
                   lax.broadcasted_iota(jnp.int32, (TN, N_GRAPHS), 1)
                   ).astype(jnp.float32)
    hh = jnp.concatenate([hT, hT * hT], axis=0)
    hi = hh.astype(jnp.bfloat16).astype(jnp.float32)
    lo = hh - hi
    s12 = (lax.dot_general(hi, oh, (((1,), (0,)), ((), ())),
                           preferred_element_type=jnp.float32)
           + lax.dot_general(lo, oh, (((1,), (0,)), ((), ())),
                             preferred_element_type=jnp.float32))
    cnt = jnp.sum(oh, axis=0)[None, :]

    @pl.when(i == 0)
    def _init():
        s12_ref[...] = s12
        cnt_ref[...] = cnt

    @pl.when(i != 0)
    def _acc():
        s12_ref[...] += s12
        cnt_ref[...] += cnt


def _tc_stats(part, w, bcol, bidx3):
    return pl.pallas_call(
        _tc_stats_body,
        grid=(NST,),
        in_specs=[
            pl.BlockSpec((NC * XD, TN // 128, 128), lambda i: (0, i, 0)),
            pl.BlockSpec((XD, EMB), lambda i: (0, 0)),
            pl.BlockSpec((EMB, 1), lambda i: (0, 0)),
            pl.BlockSpec((1, 1, TN), lambda i: (i, 0, 0)),
        ],
        out_specs=[
            pl.BlockSpec((EMB, TN // 128, 128), lambda i: (0, i, 0)),
            pl.BlockSpec((2 * EMB, N_GRAPHS), lambda i: (0, 0)),
            pl.BlockSpec((1, N_GRAPHS), lambda i: (0, 0)),
        ],
        out_shape=[
            jax.ShapeDtypeStruct((EMB, N_NODES // 128, 128), jnp.float32),
            jax.ShapeDtypeStruct((2 * EMB, N_GRAPHS), jnp.float32),
            jax.ShapeDtypeStruct((1, N_GRAPHS), jnp.float32),
        ],
    )(part, w, bcol, bidx3)


GN = N_NODES // N_GRAPHS
DB = 8
DBN = DB * GN


def _tc_dec_body(h_ref, bidx_ref, s12_ref, cnt_ref, gw_ref, gb_ref,
                 gs_ref, out_ref):
    rcnt = 1.0 / jnp.maximum(cnt_ref[...], 1.0)
    meanT = s12_ref[:EMB] * rcnt
    meansqT = s12_ref[EMB:] * rcnt
    sc = gs_ref[...]
    m2 = meanT * meanT
    varT = meansqT - (2.0 * sc) * m2 + (sc * sc) * m2
    stdT = jnp.sqrt(varT + EPS)
    scmeanT = meanT * sc
    hT = h_ref[...].reshape(EMB, DBN)
    gw = gw_ref[...]
    gb = gb_ref[...]
    for g in range(DB):
        bidx_g = bidx_ref[g, 0]
        ohT = jnp.equal(bidx_g[None, :],
                        lax.broadcasted_iota(jnp.int32, (N_GRAPHS, GN), 0)
                        ).astype(jnp.float32)
        meanT_r = lax.dot_general(scmeanT, ohT, (((1,), (0,)), ((), ())),
                                  preferred_element_type=jnp.float32)
        stdT_r = lax.dot_general(stdT, ohT, (((1,), (0,)), ((), ())),
                                 preferred_element_type=jnp.float32)
        zg = gw * (hT[:, g * GN:(g + 1) * GN] - meanT_r) / stdT_r + gb
        a = lax.dot_general(zg, zg, (((0,), (0,)), ((), ())),
                            preferred_element_type=jnp.float32)
        out_ref[g] = jax.nn.sigmoid(a)


def _tc_decode(h, bidx3, s12, cnt, gwc, gbc, gsc):
    return pl.pallas_call(
        _tc_dec_body,
        grid=(N_GRAPHS // DB,),
        in_specs=[
            pl.BlockSpec((EMB, DBN // 128, 128), lambda i: (0, i, 0)),
            pl.BlockSpec((DB, 1, GN), lambda i: (i, 0, 0)),
            pl.BlockSpec((2 * EMB, N_GRAPHS), lambda i: (0, 0)),
            pl.BlockSpec((1, N_GRAPHS), lambda i: (0, 0)),
            pl.BlockSpec((EMB, 1), lambda i: (0, 0)),
            pl.BlockSpec((EMB, 1), lambda i: (0, 0)),
            pl.BlockSpec((EMB, 1), lambda i: (0, 0)),
        ],
        out_specs=pl.BlockSpec((DB, GN, GN), lambda i: (i, 0, 0)),
        out_shape=jax.ShapeDtypeStruct((N_GRAPHS, GN, GN), jnp.float32),
    )(h, bidx3, s12, cnt, gwc, gbc, gsc)


def kernel(x, edge_index, edge_attr, batch_index, W, b, gn_weight, gn_bias,
           gn_mean_scale):
    x8 = jnp.pad(x, ((0, 0), (0, XD - IN_DIM)))
    zeros = jnp.zeros((NPT, XD), jnp.float32)
    part = _sc_aggregate(x8, edge_index, edge_attr, zeros)

    bidx_a = batch_index.reshape(NST, 1, TN)
    w8 = jnp.pad(W, ((0, XD - IN_DIM), (0, 0)))
    h, s12, cnt = _tc_stats(part, w8, b.reshape(EMB, 1), bidx_a)

    bidx_b = batch_index.reshape(N_GRAPHS, 1, GN)
    adj = _tc_decode(h, bidx_b, s12, cnt,
                     gn_weight.reshape(EMB, 1), gn_bias.reshape(EMB, 1),
                     gn_mean_scale.reshape(EMB, 1))
    return adj.reshape(N_GRAPHS, 1, GN, GN)
